# baseline (speedup 1.0000x reference)
import jax
import jax.numpy as jnp
import numpy as np
from jax.experimental import pallas as pl
from jax.experimental.pallas import tpu as pltpu

_NE = 8
_NC = 1000
_EC = 16
_GC = 8
_B = 128
_HW = 224
_OHW = 112
_NPIX = _OHW * _OHW
_M = _GC + _NE * _EC
_ME = _NE * _EC

_RG = 8
_KRG = 24
_RE = 4
_KRE = 16


def _sel_matrix():
    e = np.zeros((_HW, 384), dtype=np.float32)
    for dj in range(3):
        for j in range(_OHW):
            src = 2 * j + dj
            if src < _HW:
                e[src, 128 * dj + j] = 1.0
    return jnp.asarray(e)


def _row_onehot(r, krow):
    m = np.zeros((3, r, krow), dtype=np.float32)
    for di in range(3):
        for t in range(r):
            m[di, t, 2 * t + di] = 1.0
    return jnp.asarray(m)


def _banded(w, r, krow):
    a5 = jnp.einsum('abcd,cef->eadbf', w, _row_onehot(r, krow))
    return a5.reshape(r * w.shape[0], 9 * krow)


def _convpool_body(x_ref, e_ref, ag_ref, ae_ref, out_ref):
    xb = x_ref[0]
    xr = xb.reshape(3 * _HW, _HW)
    p3 = jnp.dot(xr, e_ref[:], preferred_element_type=jnp.float32)
    p3 = p3.reshape(3, _HW, 384)
    p3 = jnp.pad(p3, ((0, 0), (0, 16), (0, 0)))
    p3b = p3.astype(jnp.bfloat16)

    acc_g = jnp.zeros((_RG * _GC, 128), dtype=jnp.float32)
    for blk in range(_OHW // _RG):
        rows = p3[:, 16 * blk: 16 * blk + _KRG, :]
        parts = [rows[:, :, 128 * dj: 128 * (dj + 1)] for dj in range(3)]
        bb = jnp.stack(parts, axis=0).reshape(9 * _KRG, 128)
        conv = jnp.dot(ag_ref[:], bb, preferred_element_type=jnp.float32)
        acc_g = acc_g + jnp.maximum(conv, 0.0)

    acc_e = jnp.zeros((_RE * _ME, 128), dtype=jnp.float32)
    for blk in range(_OHW // _RE):
        rows = p3b[:, 8 * blk: 8 * blk + _KRE, :]
        parts = [rows[:, :, 128 * dj: 128 * (dj + 1)] for dj in range(3)]
        bb = jnp.stack(parts, axis=0).reshape(9 * _KRE, 128)
        conv = jnp.dot(ae_ref[:], bb, preferred_element_type=jnp.float32)
        acc_e = acc_e + jnp.maximum(conv, 0.0)

    pooled_g = jnp.sum(acc_g.reshape(_RG, _GC, 128), axis=(0, 2))
    pooled_e = jnp.sum(acc_e.reshape(_RE, _ME, 128), axis=(0, 2))
    out_ref[0, 0] = jnp.concatenate([pooled_g, pooled_e]) * (1.0 / _NPIX)


def _route_combine_body(pooled_ref, gwl_ref, gbl_ref, wl_ref, bl_ref,
                        out_ref, probs_ref, aux_ref):
    pooled = pooled_ref[:]
    pg = pooled[:, :_GC]
    pe = pooled[:, _GC:]
    logits = jnp.dot(pg, gwl_ref[:],
                     preferred_element_type=jnp.float32) + gbl_ref[:]
    m = jnp.max(logits, axis=1, keepdims=True)
    e = jnp.exp(logits - m)
    probs = e / jnp.sum(e, axis=1, keepdims=True)
    bw = jnp.max(probs, axis=1, keepdims=True)
    iota_e = jax.lax.broadcasted_iota(jnp.int32, (_B, _NE), 1)
    idx = jnp.min(jnp.where(probs == bw, iota_e, _NE), axis=1, keepdims=True)
    col_e = jax.lax.broadcasted_iota(jnp.int32, (_B, _NE * _EC), 1) // _EC
    z = jnp.where(col_e == idx, pe * bw, 0.0)
    onehot_w = jnp.where(iota_e == idx, bw, 0.0)
    out = (jnp.dot(z, wl_ref[:], preferred_element_type=jnp.float32)
           + jnp.dot(onehot_w, bl_ref[:], preferred_element_type=jnp.float32))
    out_ref[:] = out
    probs_ref[:] = probs
    mean_probs = jnp.mean(probs, axis=0)
    aux_ref[0, 0] = jnp.mean((mean_probs - 1.0 / _NE) ** 2)


@jax.jit
def kernel(x, gate_wc, gate_wl, gate_bl, exp_wc, exp_wl, exp_bl):
    ag = _banded(gate_wc.reshape(_GC, 3, 3, 3), _RG, _KRG)
    ae = _banded(exp_wc.reshape(_ME, 3, 3, 3), _RE, _KRE).astype(jnp.bfloat16)
    e_mat = _sel_matrix()

    pooled = pl.pallas_call(
        _convpool_body,
        grid=(_B,),
        in_specs=[
            pl.BlockSpec((1, 3, _HW, _HW), lambda b: (b, 0, 0, 0)),
            pl.BlockSpec((_HW, 384), lambda b: (0, 0)),
            pl.BlockSpec((_RG * _GC, 9 * _KRG), lambda b: (0, 0)),
            pl.BlockSpec((_RE * _ME, 9 * _KRE), lambda b: (0, 0)),
        ],
        out_specs=pl.BlockSpec((1, 1, _M), lambda b: (b, 0, 0)),
        out_shape=jax.ShapeDtypeStruct((_B, 1, _M), jnp.float32),
    )(x, e_mat, ag, ae)
    pooled = pooled.reshape(_B, _M)

    out, probs, aux = pl.pallas_call(
        _route_combine_body,
        in_specs=[pl.BlockSpec(memory_space=pltpu.VMEM)] * 5,
        out_specs=[
            pl.BlockSpec(memory_space=pltpu.VMEM),
            pl.BlockSpec(memory_space=pltpu.VMEM),
            pl.BlockSpec(memory_space=pltpu.SMEM),
        ],
        out_shape=[
            jax.ShapeDtypeStruct((_B, _NC), jnp.float32),
            jax.ShapeDtypeStruct((_B, _NE), jnp.float32),
            jax.ShapeDtypeStruct((1, 1), jnp.float32),
        ],
    )(pooled, gate_wl, gate_bl.reshape(1, _NE),
      exp_wl.reshape(_NE * _EC, _NC), exp_bl)

    return out, probs, aux.reshape(())


# speedup vs baseline: 1.0548x; 1.0548x over previous
import jax
import jax.numpy as jnp
import numpy as np
from jax.experimental import pallas as pl
from jax.experimental.pallas import tpu as pltpu

_NE = 8
_NC = 1000
_EC = 16
_GC = 8
_B = 128
_HW = 224
_OHW = 112
_NPIX = _OHW * _OHW
_M = _GC + _NE * _EC
_ME = _NE * _EC

_RG = 8
_KRG = 24
_RE = 4
_KRE = 16


def _sel_matrix():
    e = np.zeros((_HW, 384), dtype=np.float32)
    for dj in range(3):
        for j in range(_OHW):
            src = 2 * j + dj
            if src < _HW:
                e[src, 128 * dj + j] = 1.0
    return jnp.asarray(e)


def _row_onehot(r, krow):
    m = np.zeros((3, r, krow), dtype=np.float32)
    for di in range(3):
        for t in range(r):
            m[di, t, 2 * t + di] = 1.0
    return jnp.asarray(m)


def _banded(w, r, krow):
    a5 = jnp.einsum('abcd,cef->eadbf', w, _row_onehot(r, krow))
    return a5.reshape(r * w.shape[0], 9 * krow)


_G = 2


def _convpool_body(x_ref, e_ref, ag_ref, ae_ref, out_ref):
    for g in range(_G):
        xb = x_ref[g]
        xr = xb.reshape(3 * _HW, _HW)
        p3 = jnp.dot(xr, e_ref[:], preferred_element_type=jnp.float32)
        p3 = p3.reshape(3, _HW, 384)
        p3 = jnp.pad(p3, ((0, 0), (0, 16), (0, 0)))
        p3b = p3.astype(jnp.bfloat16)

        acc_g = jnp.zeros((_RG * _GC, 128), dtype=jnp.float32)
        for blk in range(_OHW // _RG):
            rows = p3[:, 16 * blk: 16 * blk + _KRG, :]
            parts = [rows[:, :, 128 * dj: 128 * (dj + 1)] for dj in range(3)]
            bb = jnp.stack(parts, axis=0).reshape(9 * _KRG, 128)
            conv = jnp.dot(ag_ref[:], bb, preferred_element_type=jnp.float32)
            acc_g = acc_g + jnp.maximum(conv, 0.0)

        acc_e = jnp.zeros((_RE * _ME, 128), dtype=jnp.float32)
        for blk in range(_OHW // _RE):
            rows = p3b[:, 8 * blk: 8 * blk + _KRE, :]
            parts = [rows[:, :, 128 * dj: 128 * (dj + 1)] for dj in range(3)]
            bb = jnp.stack(parts, axis=0).reshape(9 * _KRE, 128)
            conv = jnp.dot(ae_ref[:], bb, preferred_element_type=jnp.float32)
            acc_e = acc_e + jnp.maximum(conv, 0.0)

        pooled_g = jnp.sum(acc_g.reshape(_RG, _GC, 128), axis=(0, 2))
        pooled_e = jnp.sum(acc_e.reshape(_RE, _ME, 128), axis=(0, 2))
        out_ref[g, 0] = jnp.concatenate([pooled_g, pooled_e]) * (1.0 / _NPIX)


def _route_combine_body(pooled_ref, gwl_ref, gbl_ref, wl_ref, bl_ref,
                        out_ref, probs_ref, aux_ref):
    pooled = pooled_ref[:]
    pg = pooled[:, :_GC]
    pe = pooled[:, _GC:]
    logits = jnp.dot(pg, gwl_ref[:],
                     preferred_element_type=jnp.float32) + gbl_ref[:]
    m = jnp.max(logits, axis=1, keepdims=True)
    e = jnp.exp(logits - m)
    probs = e / jnp.sum(e, axis=1, keepdims=True)
    bw = jnp.max(probs, axis=1, keepdims=True)
    iota_e = jax.lax.broadcasted_iota(jnp.int32, (_B, _NE), 1)
    idx = jnp.min(jnp.where(probs == bw, iota_e, _NE), axis=1, keepdims=True)
    col_e = jax.lax.broadcasted_iota(jnp.int32, (_B, _NE * _EC), 1) // _EC
    z = jnp.where(col_e == idx, pe * bw, 0.0)
    onehot_w = jnp.where(iota_e == idx, bw, 0.0)
    out = (jnp.dot(z, wl_ref[:], preferred_element_type=jnp.float32)
           + jnp.dot(onehot_w, bl_ref[:], preferred_element_type=jnp.float32))
    out_ref[:] = out
    probs_ref[:] = probs
    mean_probs = jnp.mean(probs, axis=0)
    aux_ref[0, 0] = jnp.mean((mean_probs - 1.0 / _NE) ** 2)


@jax.jit
def kernel(x, gate_wc, gate_wl, gate_bl, exp_wc, exp_wl, exp_bl):
    ag = _banded(gate_wc.reshape(_GC, 3, 3, 3), _RG, _KRG)
    ae = _banded(exp_wc.reshape(_ME, 3, 3, 3), _RE, _KRE).astype(jnp.bfloat16)
    e_mat = _sel_matrix()

    pooled = pl.pallas_call(
        _convpool_body,
        grid=(_B // _G,),
        in_specs=[
            pl.BlockSpec((_G, 3, _HW, _HW), lambda b: (b, 0, 0, 0)),
            pl.BlockSpec((_HW, 384), lambda b: (0, 0)),
            pl.BlockSpec((_RG * _GC, 9 * _KRG), lambda b: (0, 0)),
            pl.BlockSpec((_RE * _ME, 9 * _KRE), lambda b: (0, 0)),
        ],
        out_specs=pl.BlockSpec((_G, 1, _M), lambda b: (b, 0, 0)),
        out_shape=jax.ShapeDtypeStruct((_B, 1, _M), jnp.float32),
    )(x, e_mat, ag, ae)
    pooled = pooled.reshape(_B, _M)

    out, probs, aux = pl.pallas_call(
        _route_combine_body,
        in_specs=[pl.BlockSpec(memory_space=pltpu.VMEM)] * 5,
        out_specs=[
            pl.BlockSpec(memory_space=pltpu.VMEM),
            pl.BlockSpec(memory_space=pltpu.VMEM),
            pl.BlockSpec(memory_space=pltpu.SMEM),
        ],
        out_shape=[
            jax.ShapeDtypeStruct((_B, _NC), jnp.float32),
            jax.ShapeDtypeStruct((_B, _NE), jnp.float32),
            jax.ShapeDtypeStruct((1, 1), jnp.float32),
        ],
    )(pooled, gate_wl, gate_bl.reshape(1, _NE),
      exp_wl.reshape(_NE * _EC, _NC), exp_bl)

    return out, probs, aux.reshape(())


# speedup vs baseline: 1.4371x; 1.3624x over previous
import jax
import jax.numpy as jnp
import numpy as np
from jax.experimental import pallas as pl
from jax.experimental.pallas import tpu as pltpu

_NE = 8
_NC = 1000
_EC = 16
_GC = 8
_B = 128
_HW = 224
_OHW = 112
_NPIX = _OHW * _OHW
_M = _GC + _NE * _EC
_ME = _NE * _EC

_RB = 8
_KR = 24
_K = 9 * _KR
_NBLK = _OHW // _RB
_G = 2
_PW = 32


def _sel_matrix():
    e = np.zeros((_HW, 256), dtype=np.float32)
    for dj in range(2):
        for j in range(_OHW):
            e[2 * j + dj, 128 * dj + j] = 1.0
    return jnp.asarray(e)


def _row_onehot():
    m = np.zeros((3, _RB, _KR), dtype=np.float32)
    for di in range(3):
        for t in range(_RB):
            m[di, t, 2 * t + di] = 1.0
    return jnp.asarray(m)


def _banded(w):
    a5 = jnp.einsum('abcd,cef->eadbf', w, _row_onehot())
    return a5.reshape(_RB * w.shape[0], _K)


def _make_bb(p3, blk):
    rows = p3[:, 16 * blk: 16 * blk + _KR, :]
    g0 = rows[:, :, 0:128]
    g1 = rows[:, :, 128:256]
    g2 = jnp.pad(g0[:, :, 1:], ((0, 0), (0, 0), (0, 1)))
    return jnp.stack([g0, g1, g2], axis=0).reshape(_K, 128)


def _convpool_body(x_ref, e_ref, ag_ref, ae_ref, gwl_ref, gbl_ref, out_ref):
    for g in range(_G):
        xb = x_ref[g]
        xr = xb.reshape(3 * _HW, _HW)
        p3 = jnp.dot(xr, e_ref[:], preferred_element_type=jnp.float32)
        p3 = p3.reshape(3, _HW, 256)
        p3 = jnp.pad(p3, ((0, 0), (0, 16), (0, 0)))

        acc_g = jnp.zeros((_RB * _GC, 128), dtype=jnp.float32)
        for blk in range(_NBLK):
            conv = jnp.dot(ag_ref[:], _make_bb(p3, blk),
                           preferred_element_type=jnp.float32)
            acc_g = acc_g + jnp.maximum(conv, 0.0)
        pooled_g = jnp.sum(acc_g.reshape(_RB, _GC, 128), axis=(0, 2)) \
            * (1.0 / _NPIX)

        lg = jnp.dot(pooled_g[None, :], gwl_ref[:],
                     preferred_element_type=jnp.float32) + gbl_ref[:]
        mx = jnp.max(lg)
        iota8 = jax.lax.broadcasted_iota(jnp.int32, (1, _NE), 1)
        idx = jnp.min(jnp.where(lg == mx, iota8, _NE))

        ae_sel = ae_ref[pl.ds(idx * _ME, _ME), :]
        acc_e = jnp.zeros((_RB * _EC, 128), dtype=jnp.float32)
        for blk in range(_NBLK):
            conv = jnp.dot(ae_sel, _make_bb(p3, blk),
                           preferred_element_type=jnp.float32)
            acc_e = acc_e + jnp.maximum(conv, 0.0)
        pooled_e = jnp.sum(acc_e.reshape(_RB, _EC, 128), axis=(0, 2)) \
            * (1.0 / _NPIX)

        out_ref[g, 0] = jnp.concatenate(
            [pooled_g, pooled_e, jnp.full((8,), idx, jnp.float32)])


def _route_combine_body(pooled_ref, gwl_ref, gbl_ref, wl_ref, bl_ref,
                        out_ref, probs_ref, aux_ref):
    pooled = pooled_ref[:]
    pg = pooled[:, :_GC]
    pe = pooled[:, _GC:_GC + _EC]
    idx = pooled[:, _GC + _EC:_GC + _EC + 1].astype(jnp.int32)
    logits = jnp.dot(pg, gwl_ref[:],
                     preferred_element_type=jnp.float32) + gbl_ref[:]
    m = jnp.max(logits, axis=1, keepdims=True)
    e = jnp.exp(logits - m)
    probs = e / jnp.sum(e, axis=1, keepdims=True)
    iota_e = jax.lax.broadcasted_iota(jnp.int32, (_B, _NE), 1)
    bw = jnp.sum(jnp.where(iota_e == idx, probs, 0.0), axis=1, keepdims=True)
    col_e = jax.lax.broadcasted_iota(jnp.int32, (_B, _ME), 1) // _EC
    val = jnp.concatenate([pe] * _NE, axis=1)
    z = jnp.where(col_e == idx, val * bw, 0.0)
    onehot_w = jnp.where(iota_e == idx, bw, 0.0)
    out = (jnp.dot(z, wl_ref[:], preferred_element_type=jnp.float32)
           + jnp.dot(onehot_w, bl_ref[:], preferred_element_type=jnp.float32))
    out_ref[:] = out
    probs_ref[:] = probs
    mean_probs = jnp.mean(probs, axis=0)
    aux_ref[0, 0] = jnp.mean((mean_probs - 1.0 / _NE) ** 2)


@jax.jit
def kernel(x, gate_wc, gate_wl, gate_bl, exp_wc, exp_wl, exp_bl):
    ag = _banded(gate_wc.reshape(_GC, 3, 3, 3))
    ae = _banded(exp_wc.reshape(_ME, 3, 3, 3))
    ae = ae.reshape(_RB, _NE, _EC, _K).transpose(1, 0, 2, 3).reshape(
        _NE * _RB * _EC, _K)
    e_mat = _sel_matrix()

    packed = pl.pallas_call(
        _convpool_body,
        grid=(_B // _G,),
        in_specs=[
            pl.BlockSpec((_G, 3, _HW, _HW), lambda b: (b, 0, 0, 0)),
            pl.BlockSpec((_HW, 256), lambda b: (0, 0)),
            pl.BlockSpec((_RB * _GC, _K), lambda b: (0, 0)),
            pl.BlockSpec((_NE * _RB * _EC, _K), lambda b: (0, 0)),
            pl.BlockSpec((_NE, _NE), lambda b: (0, 0)),
            pl.BlockSpec((1, _NE), lambda b: (0, 0)),
        ],
        out_specs=pl.BlockSpec((_G, 1, _PW), lambda b: (b, 0, 0)),
        out_shape=jax.ShapeDtypeStruct((_B, 1, _PW), jnp.float32),
    )(x, e_mat, ag, ae, gate_wl, gate_bl.reshape(1, _NE))
    packed = packed.reshape(_B, _PW)

    out, probs, aux = pl.pallas_call(
        _route_combine_body,
        in_specs=[pl.BlockSpec(memory_space=pltpu.VMEM)] * 5,
        out_specs=[
            pl.BlockSpec(memory_space=pltpu.VMEM),
            pl.BlockSpec(memory_space=pltpu.VMEM),
            pl.BlockSpec(memory_space=pltpu.SMEM),
        ],
        out_shape=[
            jax.ShapeDtypeStruct((_B, _NC), jnp.float32),
            jax.ShapeDtypeStruct((_B, _NE), jnp.float32),
            jax.ShapeDtypeStruct((1, 1), jnp.float32),
        ],
    )(packed, gate_wl, gate_bl.reshape(1, _NE),
      exp_wl.reshape(_NE * _EC, _NC), exp_bl)

    return out, probs, aux.reshape(())
